# initial kernel scaffold (unmeasured)
import jax
import jax.numpy as jnp
from jax import lax
from jax.experimental import pallas as pl
from jax.experimental.pallas import tpu as pltpu

N_DEV = 4
M_PER = 3072
K = 1536
N = 3072
TILE_M = 1024
HALF = 1536


def kernel(A, B):
    def body(a_hbm, b_hbm, out_hbm, staging, b16, comm, ctile,
             load_sem, store_sem, send_sems, recv_sems):
        my = lax.axis_index("i")
        left = (my + N_DEV - 1) % N_DEV
        right = (my + 1) % N_DEV

        for t in range(2):
            cp = pltpu.make_async_copy(
                b_hbm.at[:, pl.ds(t * HALF, HALF)], staging, load_sem)
            cp.start()
            cp.wait()
            b16[:, pl.ds(t * HALF, HALF)] = staging[...].astype(jnp.bfloat16)
        for t in range(2):
            cp = pltpu.make_async_copy(
                a_hbm.at[pl.ds(t * HALF, HALF), :], staging, load_sem)
            cp.start()
            cp.wait()
            comm[0, pl.ds(t * HALF, HALF), :] = staging[...].astype(jnp.bfloat16)

        barrier_sem = pltpu.get_barrier_semaphore()
        for nbr in [left, right]:
            pl.semaphore_signal(
                barrier_sem, inc=1,
                device_id=(nbr,), device_id_type=pl.DeviceIdType.MESH)
        pl.semaphore_wait(barrier_sem, 2)

        def compute_chunk(slot, origin):
            for t in range(M_PER // TILE_M):
                ctile[...] = jnp.dot(
                    comm[slot, t * TILE_M:(t + 1) * TILE_M, :], b16[...],
                    preferred_element_type=jnp.float32)
                st = pltpu.make_async_copy(
                    ctile,
                    out_hbm.at[pl.ds(origin * M_PER + t * TILE_M, TILE_M), :],
                    store_sem)
                st.start()
                st.wait()

        for h in range(N_DEV - 1):
            s, r = h % 2, (h + 1) % 2
            rdma = pltpu.make_async_remote_copy(
                src_ref=comm.at[s],
                dst_ref=comm.at[r],
                send_sem=send_sems.at[h],
                recv_sem=recv_sems.at[h],
                device_id=(right,),
                device_id_type=pl.DeviceIdType.MESH)
            rdma.start()
            compute_chunk(s, (my + N_DEV - h) % N_DEV)
            rdma.wait()
        compute_chunk((N_DEV - 1) % 2, (my + 1) % N_DEV)

    return pl.pallas_call(
        body,
        out_shape=jax.ShapeDtypeStruct((N_DEV * M_PER, N), jnp.float32),
        in_specs=[
            pl.BlockSpec(memory_space=pltpu.MemorySpace.HBM),
            pl.BlockSpec(memory_space=pltpu.MemorySpace.HBM),
        ],
        out_specs=pl.BlockSpec(memory_space=pltpu.MemorySpace.HBM),
        scratch_shapes=[
            pltpu.VMEM((HALF, HALF), jnp.float32),
            pltpu.VMEM((K, N), jnp.bfloat16),
            pltpu.VMEM((2, M_PER, K), jnp.bfloat16),
            pltpu.VMEM((TILE_M, N), jnp.float32),
            pltpu.SemaphoreType.DMA,
            pltpu.SemaphoreType.DMA,
            pltpu.SemaphoreType.DMA((N_DEV - 1,)),
            pltpu.SemaphoreType.DMA((N_DEV - 1,)),
        ],
        compiler_params=pltpu.CompilerParams(collective_id=0),
    )(A, B)


# baseline (device time: 470236 ns/iter reference)
import jax
import jax.numpy as jnp
from jax import lax
from jax.experimental import pallas as pl
from jax.experimental.pallas import tpu as pltpu

N_DEV = 4
M_PER = 3072
K = 1536
N = 3072
TILE_M = 1024
HALF = 1536


def kernel(A, B):
    def body(a_hbm, b_hbm, out_hbm, staging, b16, comm, ctile,
             load_sem, store_sem, send_sems, recv_sems):
        my = lax.axis_index("i")
        left = (my + N_DEV - 1) % N_DEV
        right = (my + 1) % N_DEV

        for t in range(2):
            cp = pltpu.make_async_copy(
                b_hbm.at[:, pl.ds(t * HALF, HALF)], staging, load_sem)
            cp.start()
            cp.wait()
            b16[:, pl.ds(t * HALF, HALF)] = staging[...].astype(jnp.bfloat16)
        for t in range(2):
            cp = pltpu.make_async_copy(
                a_hbm.at[pl.ds(t * HALF, HALF), :], staging, load_sem)
            cp.start()
            cp.wait()
            comm[0, pl.ds(t * HALF, HALF), :] = staging[...].astype(jnp.bfloat16)

        barrier_sem = pltpu.get_barrier_semaphore()
        for nbr in [left, right]:
            pl.semaphore_signal(
                barrier_sem, inc=1,
                device_id=(nbr,), device_id_type=pl.DeviceIdType.MESH)
        pl.semaphore_wait(barrier_sem, 2)

        def compute_chunk(slot, origin):
            for t in range(M_PER // TILE_M):
                ctile[...] = jnp.dot(
                    comm[slot, t * TILE_M:(t + 1) * TILE_M, :], b16[...],
                    preferred_element_type=jnp.float32)
                st = pltpu.make_async_copy(
                    ctile,
                    out_hbm.at[pl.ds(origin * M_PER + t * TILE_M, TILE_M), :],
                    store_sem)
                st.start()
                st.wait()

        for h in range(N_DEV - 1):
            s, r = h % 2, (h + 1) % 2
            rdma = pltpu.make_async_remote_copy(
                src_ref=comm.at[s],
                dst_ref=comm.at[r],
                send_sem=send_sems.at[h],
                recv_sem=recv_sems.at[h],
                device_id=(right,),
                device_id_type=pl.DeviceIdType.MESH)
            rdma.start()
            compute_chunk(s, (my + N_DEV - h) % N_DEV)
            rdma.wait()
        compute_chunk((N_DEV - 1) % 2, (my + 1) % N_DEV)

    return pl.pallas_call(
        body,
        out_shape=jax.ShapeDtypeStruct((N_DEV * M_PER, N), jnp.float32),
        in_specs=[
            pl.BlockSpec(memory_space=pltpu.MemorySpace.HBM),
            pl.BlockSpec(memory_space=pltpu.MemorySpace.HBM),
        ],
        out_specs=pl.BlockSpec(memory_space=pltpu.MemorySpace.HBM),
        scratch_shapes=[
            pltpu.VMEM((HALF, HALF), jnp.float32),
            pltpu.VMEM((K, N), jnp.bfloat16),
            pltpu.VMEM((2, M_PER, K), jnp.bfloat16),
            pltpu.VMEM((TILE_M, N), jnp.float32),
            pltpu.SemaphoreType.DMA,
            pltpu.SemaphoreType.DMA,
            pltpu.SemaphoreType.DMA((N_DEV - 1,)),
            pltpu.SemaphoreType.DMA((N_DEV - 1,)),
        ],
        compiler_params=pltpu.CompilerParams(
            collective_id=0, vmem_limit_bytes=60 * 1024 * 1024),
    )(A, B)


# device time: 309331 ns/iter; 1.5202x vs baseline; 1.5202x over previous
import jax
import jax.numpy as jnp
from jax import lax
from jax.experimental import pallas as pl
from jax.experimental.pallas import tpu as pltpu

N_DEV = 4
M_PER = 3072
K = 1536
N = 3072
HALF = 1536


def kernel(A, B):
    def body(a_hbm, b_hbm, out_hbm, staging, b16, comm_r, comm_l, ctile,
             load_sem, store_sem, send_r, recv_r, send_l, recv_l):
        my = lax.axis_index("i")
        left = (my + N_DEV - 1) % N_DEV
        right = (my + 1) % N_DEV

        def load_half(src_slice, dst_ref):
            cp = pltpu.make_async_copy(src_slice, staging, load_sem)
            cp.start()
            cp.wait()
            dst_ref[...] = staging[...].astype(jnp.bfloat16)

        load_half(a_hbm.at[pl.ds(0, HALF), :], comm_r.at[0])
        load_half(a_hbm.at[pl.ds(HALF, HALF), :], comm_l.at[0])

        barrier_sem = pltpu.get_barrier_semaphore()
        for nbr in [left, right]:
            pl.semaphore_signal(
                barrier_sem, inc=1,
                device_id=(nbr,), device_id_type=pl.DeviceIdType.MESH)
        pl.semaphore_wait(barrier_sem, 2)

        def compute_half(comm, slot, origin, row_off):
            ctile[...] = jnp.dot(comm[slot], b16[...],
                                 preferred_element_type=jnp.float32)
            st = pltpu.make_async_copy(
                ctile,
                out_hbm.at[pl.ds(origin * M_PER + row_off, HALF), :],
                store_sem)
            st.start()
            st.wait()

        for h in range(N_DEV - 1):
            s, r = h % 2, (h + 1) % 2
            rdma_r = pltpu.make_async_remote_copy(
                src_ref=comm_r.at[s], dst_ref=comm_r.at[r],
                send_sem=send_r.at[h], recv_sem=recv_r.at[h],
                device_id=(right,), device_id_type=pl.DeviceIdType.MESH)
            rdma_l = pltpu.make_async_remote_copy(
                src_ref=comm_l.at[s], dst_ref=comm_l.at[r],
                send_sem=send_l.at[h], recv_sem=recv_l.at[h],
                device_id=(left,), device_id_type=pl.DeviceIdType.MESH)
            rdma_r.start()
            rdma_l.start()
            if h == 0:
                load_half(b_hbm.at[:, pl.ds(0, HALF)],
                          b16.at[:, pl.ds(0, HALF)])
                load_half(b_hbm.at[:, pl.ds(HALF, HALF)],
                          b16.at[:, pl.ds(HALF, HALF)])
            compute_half(comm_r, s, (my + N_DEV - h) % N_DEV, 0)
            compute_half(comm_l, s, (my + h) % N_DEV, HALF)
            rdma_r.wait()
            rdma_l.wait()
        last = (N_DEV - 1) % 2
        compute_half(comm_r, last, (my + 1) % N_DEV, 0)
        compute_half(comm_l, last, (my + N_DEV - 1) % N_DEV, HALF)

    return pl.pallas_call(
        body,
        out_shape=jax.ShapeDtypeStruct((N_DEV * M_PER, N), jnp.float32),
        in_specs=[
            pl.BlockSpec(memory_space=pltpu.MemorySpace.HBM),
            pl.BlockSpec(memory_space=pltpu.MemorySpace.HBM),
        ],
        out_specs=pl.BlockSpec(memory_space=pltpu.MemorySpace.HBM),
        scratch_shapes=[
            pltpu.VMEM((HALF, HALF), jnp.float32),
            pltpu.VMEM((K, N), jnp.bfloat16),
            pltpu.VMEM((2, HALF, K), jnp.bfloat16),
            pltpu.VMEM((2, HALF, K), jnp.bfloat16),
            pltpu.VMEM((HALF, N), jnp.float32),
            pltpu.SemaphoreType.DMA,
            pltpu.SemaphoreType.DMA,
            pltpu.SemaphoreType.DMA((N_DEV - 1,)),
            pltpu.SemaphoreType.DMA((N_DEV - 1,)),
            pltpu.SemaphoreType.DMA((N_DEV - 1,)),
            pltpu.SemaphoreType.DMA((N_DEV - 1,)),
        ],
        compiler_params=pltpu.CompilerParams(
            collective_id=0, vmem_limit_bytes=60 * 1024 * 1024),
    )(A, B)
